# Initial kernel scaffold; baseline (speedup 1.0000x reference)
import jax
import jax.numpy as jnp
from jax import lax
from jax.experimental import pallas as pl
from jax.experimental.pallas import tpu as pltpu
from jax.experimental.pallas import tpu_sc as plsc

VOCAB = 128
D = 128
B = 16384
NC = 2
NS = 16
NW = NC * NS
BPW = B // NW
CHUNK = 128
NCHUNK = BPW // CHUNK


def _body(pos_hbm, row_hbm, col_hbm, out_hbm,
          pos_v, ridx_v, cidx_v, rows_v, cols_v, sem_r, sem_c):
    wid = lax.axis_index("s") * NC + lax.axis_index("c")
    base = wid * BPW
    pltpu.sync_copy(pos_hbm.at[pl.ds(base * 4, BPW * 4)], pos_v)
    lanes = lax.iota(jnp.int32, 16)

    for c in range(NCHUNK):
        def idx_body(j, carry):
            i4 = (c * CHUNK + j * 16 + lanes) * 4
            pr0 = plsc.load_gather(pos_v, [i4])
            pc0 = plsc.load_gather(pos_v, [i4 + 1])
            pr1 = plsc.load_gather(pos_v, [i4 + 2])
            pc1 = plsc.load_gather(pos_v, [i4 + 3])
            qr0 = jnp.minimum((pr0 * VOCAB).astype(jnp.int32), VOCAB - 1)
            qr1 = jnp.minimum((pr1 * VOCAB).astype(jnp.int32), VOCAB - 1)
            qc0 = jnp.minimum((pc0 * VOCAB).astype(jnp.int32), VOCAB - 1)
            qc1 = jnp.minimum((pc1 * VOCAB).astype(jnp.int32), VOCAB - 1)
            ridx_v[pl.ds(j * 16, 16)] = jnp.right_shift(qr0 + qr1, 1)
            cidx_v[pl.ds(j * 16, 16)] = jnp.right_shift(qc0 + qc1, 1)
            return carry
        lax.fori_loop(0, CHUNK // 16, idx_body, 0)

        cp_r = pltpu.async_copy(row_hbm.at[ridx_v], rows_v, sem_r)
        cp_c = pltpu.async_copy(col_hbm.at[cidx_v], cols_v, sem_c)
        cp_r.wait()
        cp_c.wait()

        def add_body(r, carry):
            for k in range(D // 16):
                s = pl.ds(k * 16, 16)
                rows_v[r, s] = rows_v[r, s] + cols_v[r, s]
            return carry
        lax.fori_loop(0, CHUNK, add_body, 0)

        pltpu.sync_copy(rows_v, out_hbm.at[pl.ds(base + c * CHUNK, CHUNK)])


_mesh = plsc.VectorSubcoreMesh(core_axis_name="c", subcore_axis_name="s")

_kern = pl.kernel(
    _body,
    out_type=jax.ShapeDtypeStruct((B, D), jnp.float32),
    mesh=_mesh,
    scratch_types=[
        pltpu.VMEM((BPW * 4,), jnp.float32),
        pltpu.VMEM((CHUNK,), jnp.int32),
        pltpu.VMEM((CHUNK,), jnp.int32),
        pltpu.VMEM((CHUNK, D), jnp.float32),
        pltpu.VMEM((CHUNK, D), jnp.float32),
        pltpu.SemaphoreType.DMA,
        pltpu.SemaphoreType.DMA,
    ],
)


def kernel(patch_positions, row_embedding, column_embedding):
    pos_flat = patch_positions.reshape(B * 2 * 2)
    return _kern(pos_flat, row_embedding, column_embedding)


# speedup vs baseline: 2.2266x; 2.2266x over previous
import jax
import jax.numpy as jnp
from jax import lax
from jax.experimental import pallas as pl
from jax.experimental.pallas import tpu as pltpu
from jax.experimental.pallas import tpu_sc as plsc

VOCAB = 128
D = 128
B = 16384
NC = 2
NS = 16
NW = NC * NS
BPW = B // NW
CHUNK = 128
NCHUNK = BPW // CHUNK


def _body(pos_hbm, row_hbm, col_hbm, out_hbm,
          pos_v, ridx_v, cidx_v, rows_v, cols_v, sem_r, sem_c):
    wid = lax.axis_index("s") * NC + lax.axis_index("c")
    base = wid * BPW
    for a in range(4):
        pltpu.sync_copy(pos_hbm.at[a, pl.ds(base, BPW)], pos_v.at[a])

    for c in range(NCHUNK):
        def idx_body(j, carry):
            s = pl.ds(c * CHUNK + j * 16, 16)
            pr0 = pos_v[0, s]
            pc0 = pos_v[1, s]
            pr1 = pos_v[2, s]
            pc1 = pos_v[3, s]
            qr0 = jnp.minimum((pr0 * VOCAB).astype(jnp.int32), VOCAB - 1)
            qr1 = jnp.minimum((pr1 * VOCAB).astype(jnp.int32), VOCAB - 1)
            qc0 = jnp.minimum((pc0 * VOCAB).astype(jnp.int32), VOCAB - 1)
            qc1 = jnp.minimum((pc1 * VOCAB).astype(jnp.int32), VOCAB - 1)
            ridx_v[pl.ds(j * 16, 16)] = jnp.right_shift(qr0 + qr1, 1)
            cidx_v[pl.ds(j * 16, 16)] = jnp.right_shift(qc0 + qc1, 1)
            return carry
        lax.fori_loop(0, CHUNK // 16, idx_body, 0)

        cp_r = pltpu.async_copy(row_hbm.at[ridx_v], rows_v, sem_r)
        cp_c = pltpu.async_copy(col_hbm.at[cidx_v], cols_v, sem_c)
        cp_r.wait()
        cp_c.wait()

        def add_body(r, carry):
            for k in range(D // 16):
                s = pl.ds(k * 16, 16)
                rows_v[r, s] = rows_v[r, s] + cols_v[r, s]
            return carry
        lax.fori_loop(0, CHUNK, add_body, 0)

        pltpu.sync_copy(rows_v, out_hbm.at[pl.ds(base + c * CHUNK, CHUNK)])


_mesh = plsc.VectorSubcoreMesh(core_axis_name="c", subcore_axis_name="s")

_kern = pl.kernel(
    _body,
    out_type=jax.ShapeDtypeStruct((B, D), jnp.float32),
    mesh=_mesh,
    scratch_types=[
        pltpu.VMEM((4, BPW), jnp.float32),
        pltpu.VMEM((CHUNK,), jnp.int32),
        pltpu.VMEM((CHUNK,), jnp.int32),
        pltpu.VMEM((CHUNK, D), jnp.float32),
        pltpu.VMEM((CHUNK, D), jnp.float32),
        pltpu.SemaphoreType.DMA,
        pltpu.SemaphoreType.DMA,
    ],
)


def kernel(patch_positions, row_embedding, column_embedding):
    pos_planes = patch_positions.reshape(B, 4).T
    return _kern(pos_planes, row_embedding, column_embedding)


# speedup vs baseline: 2.2657x; 1.0176x over previous
import jax
import jax.numpy as jnp
from jax import lax
from jax.experimental import pallas as pl
from jax.experimental.pallas import tpu as pltpu
from jax.experimental.pallas import tpu_sc as plsc

VOCAB = 128
D = 128
B = 16384
NC = 2
NS = 16
NW = NC * NS
BPW = B // NW
CHUNK = 128
NCHUNK = BPW // CHUNK


def _body(pos_hbm, row_hbm, col_hbm, out_hbm,
          pos_v, rtab_v, ctab_v, ridx_v, cidx_v, out_v, sem_in, sem_out):
    wid = lax.axis_index("s") * NC + lax.axis_index("c")
    base = wid * BPW

    cps = [pltpu.async_copy(row_hbm, rtab_v, sem_in),
           pltpu.async_copy(col_hbm, ctab_v, sem_in)]
    for a in range(4):
        cps.append(
            pltpu.async_copy(pos_hbm.at[a, pl.ds(base, BPW)], pos_v.at[a],
                             sem_in))

    for cp in cps:
        cp.wait()

    def idx_body(j, carry):
        s = pl.ds(j * 16, 16)
        qr0 = jnp.minimum((pos_v[0, s] * VOCAB).astype(jnp.int32), VOCAB - 1)
        qc0 = jnp.minimum((pos_v[1, s] * VOCAB).astype(jnp.int32), VOCAB - 1)
        qr1 = jnp.minimum((pos_v[2, s] * VOCAB).astype(jnp.int32), VOCAB - 1)
        qc1 = jnp.minimum((pos_v[3, s] * VOCAB).astype(jnp.int32), VOCAB - 1)
        ridx_v[s] = jnp.right_shift(qr0 + qr1, 1)
        cidx_v[s] = jnp.right_shift(qc0 + qc1, 1)
        return carry
    lax.fori_loop(0, BPW // 16, idx_body, 0)

    out_cps = [None, None]
    for c in range(NCHUNK):
        buf = c % 2
        if out_cps[buf] is not None:
            out_cps[buf].wait()

        def row_body(g, carry):
            rvec = ridx_v[pl.ds(c * CHUNK + g * 16, 16)]
            cvec = cidx_v[pl.ds(c * CHUNK + g * 16, 16)]
            for e in range(16):
                ri = rvec[e]
                ci = cvec[e]
                for k in range(D // 16):
                    s = pl.ds(k * 16, 16)
                    out_v[buf, g * 16 + e, s] = rtab_v[ri, s] + ctab_v[ci, s]
            return carry
        lax.fori_loop(0, CHUNK // 16, row_body, 0)

        out_cps[buf] = pltpu.async_copy(
            out_v.at[buf], out_hbm.at[pl.ds(base + c * CHUNK, CHUNK)],
            sem_out)

    for cp in out_cps:
        if cp is not None:
            cp.wait()


_mesh = plsc.VectorSubcoreMesh(core_axis_name="c", subcore_axis_name="s")

_kern = pl.kernel(
    _body,
    out_type=jax.ShapeDtypeStruct((B, D), jnp.float32),
    mesh=_mesh,
    scratch_types=[
        pltpu.VMEM((4, BPW), jnp.float32),
        pltpu.VMEM((VOCAB, D), jnp.float32),
        pltpu.VMEM((VOCAB, D), jnp.float32),
        pltpu.VMEM((BPW,), jnp.int32),
        pltpu.VMEM((BPW,), jnp.int32),
        pltpu.VMEM((2, CHUNK, D), jnp.float32),
        pltpu.SemaphoreType.DMA,
        pltpu.SemaphoreType.DMA,
    ],
)


def kernel(patch_positions, row_embedding, column_embedding):
    pos_planes = patch_positions.reshape(B, 4).T
    return _kern(pos_planes, row_embedding, column_embedding)


# speedup vs baseline: 2.8325x; 1.2502x over previous
import jax
import jax.numpy as jnp
from jax import lax
from jax.experimental import pallas as pl
from jax.experimental.pallas import tpu as pltpu
from jax.experimental.pallas import tpu_sc as plsc

VOCAB = 128
D = 128
B = 16384
NC = 2
NS = 16
NW = NC * NS
BPW = B // NW
CHUNK = 256
NCHUNK = BPW // CHUNK


def _body(pos_hbm, row_hbm, col_hbm, out_hbm,
          pos_v, rtab_v, ctab_v, ridx_v, cidx_v, out_v, sem_in, sem_out):
    wid = lax.axis_index("s") * NC + lax.axis_index("c")
    base = wid * BPW

    cps = [pltpu.async_copy(row_hbm, rtab_v, sem_in),
           pltpu.async_copy(col_hbm, ctab_v, sem_in)]
    for a in range(4):
        cps.append(
            pltpu.async_copy(pos_hbm.at[a, pl.ds(base, BPW)], pos_v.at[a],
                             sem_in))

    for cp in cps:
        cp.wait()

    @plsc.parallel_loop(0, BPW // 16)
    def idx_body(j):
        s = pl.ds(j * 16, 16)
        qr0 = jnp.minimum((pos_v[0, s] * VOCAB).astype(jnp.int32), VOCAB - 1)
        qc0 = jnp.minimum((pos_v[1, s] * VOCAB).astype(jnp.int32), VOCAB - 1)
        qr1 = jnp.minimum((pos_v[2, s] * VOCAB).astype(jnp.int32), VOCAB - 1)
        qc1 = jnp.minimum((pos_v[3, s] * VOCAB).astype(jnp.int32), VOCAB - 1)
        ridx_v[s] = jnp.right_shift(qr0 + qr1, 1)
        cidx_v[s] = jnp.right_shift(qc0 + qc1, 1)

    out_cps = [None, None]
    for c in range(NCHUNK):
        buf = c % 2
        if out_cps[buf] is not None:
            out_cps[buf].wait()

        @plsc.parallel_loop(0, CHUNK // 16)
        def row_body(g):
            rvec = ridx_v[pl.ds(c * CHUNK + g * 16, 16)]
            cvec = cidx_v[pl.ds(c * CHUNK + g * 16, 16)]
            for e in range(16):
                ri = rvec[e]
                ci = cvec[e]
                for k in range(D // 16):
                    s = pl.ds(k * 16, 16)
                    out_v[buf, g * 16 + e, s] = rtab_v[ri, s] + ctab_v[ci, s]

        out_cps[buf] = pltpu.async_copy(
            out_v.at[buf], out_hbm.at[pl.ds(base + c * CHUNK, CHUNK)],
            sem_out)

    for cp in out_cps:
        if cp is not None:
            cp.wait()


_mesh = plsc.VectorSubcoreMesh(core_axis_name="c", subcore_axis_name="s")

_kern = pl.kernel(
    _body,
    out_type=jax.ShapeDtypeStruct((B, D), jnp.float32),
    mesh=_mesh,
    scratch_types=[
        pltpu.VMEM((4, BPW), jnp.float32),
        pltpu.VMEM((VOCAB, D), jnp.float32),
        pltpu.VMEM((VOCAB, D), jnp.float32),
        pltpu.VMEM((BPW,), jnp.int32),
        pltpu.VMEM((BPW,), jnp.int32),
        pltpu.VMEM((2, CHUNK, D), jnp.float32),
        pltpu.SemaphoreType.DMA,
        pltpu.SemaphoreType.DMA,
    ],
)


def kernel(patch_positions, row_embedding, column_embedding):
    pos_planes = patch_positions.reshape(B, 4).T
    return _kern(pos_planes, row_embedding, column_embedding)
